# baseline (speedup 1.0000x reference)
import functools
import math

import jax
import jax.numpy as jnp
import numpy as np
from jax import lax
from jax.experimental import pallas as pl
from jax.experimental.pallas import tpu as pltpu
from jax.experimental.pallas import tpu_sc as plsc

_HIDDEN = 128
_MAX_LEN = 512


def _pe_table(d_model, max_len):
    pe = np.zeros((max_len, d_model), dtype=np.float32)
    position = np.arange(0, max_len, dtype=np.float32)[:, None]
    div_term = np.exp(
        np.arange(0, d_model, 2, dtype=np.float32) * -(math.log(10000.0) / d_model)
    )
    pe[:, 0::2] = np.sin(position * div_term)
    pe[:, 1::2] = np.cos(position * div_term)
    return pe


_PE = _pe_table(_HIDDEN, _MAX_LEN)

_NUM_CORES = 2
_NUM_SUBCORES = 16
_NW = _NUM_CORES * _NUM_SUBCORES
_LANES = 16
_NBUF = 3


@functools.lru_cache(maxsize=None)
def _build(B, L, V, D):
    nseq = B // _NW
    n_outer = nseq // _NBUF
    n_tail = nseq - _NBUF * n_outer
    assert n_tail == _NBUF - 1
    c1 = min(L, 128)
    c2 = L - c1
    mesh = plsc.VectorSubcoreMesh(core_axis_name="c", subcore_axis_name="s")

    @functools.partial(
        pl.kernel,
        out_type=jax.ShapeDtypeStruct((B * L, D), jnp.float32),
        mesh=mesh,
        scratch_types=[
            [pltpu.VMEM((L,), jnp.int32)] * _NBUF,
            [pltpu.VMEM((L, D), jnp.float32)] * _NBUF,
            pltpu.VMEM((L, D), jnp.float32),
            pltpu.VMEM_SHARED((V, D), jnp.float32),
            [pltpu.SemaphoreType.DMA] * _NBUF,
            [pltpu.SemaphoreType.DMA] * _NBUF,
        ],
    )
    def emb_kernel(
        x_hbm, pe_hbm, t_hbm, out_hbm,
        idxs, rows, pe_v, t_sp, gsems, ssems,
    ):
        sid = lax.axis_index("s")
        wid = sid * _NUM_CORES + lax.axis_index("c")
        base0 = wid * nseq * L

        def chunks(buf, sem):
            return (
                pltpu.make_async_copy(
                    t_sp.at[idxs[buf].at[pl.ds(0, c1)]],
                    rows[buf].at[pl.ds(0, c1)],
                    sem,
                ),
                pltpu.make_async_copy(
                    t_sp.at[idxs[buf].at[pl.ds(c1, c2)]],
                    rows[buf].at[pl.ds(c1, c2)],
                    sem,
                ),
            )

        def start_gather(buf, seq_base):
            pltpu.sync_copy(x_hbm.at[pl.ds(seq_base, L)], idxs[buf])
            for c in chunks(buf, gsems[buf]):
                c.start()

        def wait_gather(buf):
            for c in chunks(buf, gsems[buf]):
                c.wait()

        def wait_store(buf, seq_base):
            pltpu.make_async_copy(
                rows[buf], out_hbm.at[pl.ds(seq_base, L)], ssems[buf]
            ).wait()

        n_stage = 5
        per_stage = V // n_stage

        @pl.when(sid < n_stage)
        def _():
            pltpu.sync_copy(
                t_hbm.at[pl.ds(sid * per_stage, per_stage)],
                t_sp.at[pl.ds(sid * per_stage, per_stage)],
            )

        pltpu.sync_copy(pe_hbm, pe_v)
        plsc.subcore_barrier()
        for b in range(_NBUF - 1):
            start_gather(b, base0 + b * L)

        def add_and_store(b, base):
            @plsc.parallel_loop(0, L, 2, unroll=4)
            def _(r):
                for rr in range(2):
                    for cc in range(D // _LANES):
                        sl = pl.ds(cc * _LANES, _LANES)
                        plsc.addupdate(
                            rows[b].at[r + rr, sl], pe_v[r + rr, sl]
                        )

            pltpu.async_copy(rows[b], out_hbm.at[pl.ds(base, L)], ssems[b])

        def body(i, carry):
            for b in range(_NBUF):
                s = _NBUF * i + b
                base = base0 + s * L
                wait_gather(b)
                pb = (b + _NBUF - 1) % _NBUF
                if b == 0:

                    @pl.when(s > 0)
                    def _():
                        wait_store(pb, base - L)

                else:
                    wait_store(pb, base - L)
                start_gather(pb, base + (_NBUF - 1) * L)
                add_and_store(b, base)
            return carry

        lax.fori_loop(0, n_outer, body, 0)

        for t in range(n_tail):
            s = _NBUF * n_outer + t
            wait_gather(s % _NBUF)
            add_and_store(s % _NBUF, base0 + s * L)

        for t in range(_NBUF):
            s = nseq - _NBUF + t
            wait_store(s % _NBUF, base0 + s * L)

    return emb_kernel


def kernel(x, pos_num, table):
    B, L = x.shape
    V, D = table.shape
    t = table.at[0].set(0.0)
    pe = jnp.asarray(_PE[:L])
    xf = x.reshape(B * L).astype(jnp.int32)
    out = _build(B, L, V, D)(xf, pe, t)
    return out.reshape(B, L, D)


# speedup vs baseline: 1.3134x; 1.3134x over previous
import functools
import math

import jax
import jax.numpy as jnp
import numpy as np
from jax import lax
from jax.experimental import pallas as pl
from jax.experimental.pallas import tpu as pltpu
from jax.experimental.pallas import tpu_sc as plsc

_HIDDEN = 128
_MAX_LEN = 512


def _pe_table(d_model, max_len):
    pe = np.zeros((max_len, d_model), dtype=np.float32)
    position = np.arange(0, max_len, dtype=np.float32)[:, None]
    div_term = np.exp(
        np.arange(0, d_model, 2, dtype=np.float32) * -(math.log(10000.0) / d_model)
    )
    pe[:, 0::2] = np.sin(position * div_term)
    pe[:, 1::2] = np.cos(position * div_term)
    return pe


_PE = _pe_table(_HIDDEN, _MAX_LEN)

_NUM_CORES = 2
_NUM_SUBCORES = 16
_NW = _NUM_CORES * _NUM_SUBCORES
_NBUF = 4


@functools.lru_cache(maxsize=None)
def _build(B, L, V, D):
    nseq = B // _NW
    n_outer = nseq // _NBUF
    assert nseq == _NBUF * n_outer
    c1 = min(L, 128)
    c2 = L - c1
    mesh = plsc.VectorSubcoreMesh(core_axis_name="c", subcore_axis_name="s")

    @functools.partial(
        pl.kernel,
        out_type=jax.ShapeDtypeStruct((B * L, D), jnp.float32),
        mesh=mesh,
        scratch_types=[
            pltpu.VMEM((nseq, c1), jnp.int32),
            pltpu.VMEM((nseq, c2), jnp.int32),
            pltpu.VMEM((L,), jnp.int32),
            [pltpu.VMEM((L, D), jnp.float32)] * _NBUF,
            pltpu.VMEM_SHARED((V, D), jnp.float32),
            pltpu.VMEM_SHARED((L, D), jnp.float32),
            [pltpu.SemaphoreType.DMA] * _NBUF,
            [pltpu.SemaphoreType.DMA] * _NBUF,
            [pltpu.SemaphoreType.DMA] * _NBUF,
        ],
    )
    def emb_kernel(
        xa_hbm, xb_hbm, pe_hbm, t_hbm, iota_hbm, out_hbm,
        idx_a, idx_b, idx_pe, rows, t_sp, pe_sp, gsems, asems, ssems,
    ):
        sid = lax.axis_index("s")
        wid = sid * _NUM_CORES + lax.axis_index("c")
        base0 = wid * nseq * L

        def gather_copies(k, s):
            return (
                pltpu.make_async_copy(
                    t_sp.at[idx_a.at[s]], rows[k].at[pl.ds(0, c1)], gsems[k]
                ),
                pltpu.make_async_copy(
                    t_sp.at[idx_b.at[s]], rows[k].at[pl.ds(c1, c2)], gsems[k]
                ),
            )

        def pe_copies(k):
            return (
                pltpu.make_async_copy(
                    pe_sp.at[idx_pe.at[pl.ds(0, c1)]],
                    rows[k].at[pl.ds(0, c1)],
                    asems[k],
                ),
                pltpu.make_async_copy(
                    pe_sp.at[idx_pe.at[pl.ds(c1, c2)]],
                    rows[k].at[pl.ds(c1, c2)],
                    asems[k],
                ),
            )

        def start_pe(k):
            pltpu.async_copy(
                pe_sp.at[idx_pe.at[pl.ds(0, c1)]],
                rows[k].at[pl.ds(0, c1)],
                asems[k],
                add=True,
            )
            pltpu.async_copy(
                pe_sp.at[idx_pe.at[pl.ds(c1, c2)]],
                rows[k].at[pl.ds(c1, c2)],
                asems[k],
                add=True,
            )

        def store_copy(k, s):
            return pltpu.make_async_copy(
                rows[k], out_hbm.at[pl.ds(base0 + s * L, L)], ssems[k]
            )

        n_stage = 5
        per_stage = V // n_stage

        @pl.when(sid < n_stage)
        def _():
            pltpu.sync_copy(
                t_hbm.at[pl.ds(sid * per_stage, per_stage)],
                t_sp.at[pl.ds(sid * per_stage, per_stage)],
            )

        @pl.when(sid == n_stage)
        def _():
            pltpu.sync_copy(pe_hbm, pe_sp)

        pltpu.sync_copy(xa_hbm.at[wid], idx_a)
        pltpu.sync_copy(xb_hbm.at[wid], idx_b)
        pltpu.sync_copy(iota_hbm, idx_pe)
        plsc.subcore_barrier()
        for k in range(_NBUF - 1):
            for c in gather_copies(k, k):
                c.start()

        def body(i, carry):
            for k in range(_NBUF):
                s = _NBUF * i + k
                for c in gather_copies(k, s):
                    c.wait()
                start_pe(k)
                pk = (k + _NBUF - 1) % _NBUF
                if k == 0:

                    @pl.when(s > 0)
                    def _():
                        store_copy(pk, s - 1).wait()

                    for c in gather_copies(pk, s + _NBUF - 1):
                        c.start()
                else:

                    @pl.when(i < n_outer - 1)
                    def _():
                        store_copy(pk, s - 1).wait()
                        for c in gather_copies(pk, s + _NBUF - 1):
                            c.start()

                for c in pe_copies(k):
                    c.wait()
                store_copy(k, s).start()
            return carry

        lax.fori_loop(0, n_outer, body, 0)

        for k in range(_NBUF):
            s = nseq - _NBUF + k
            store_copy(s % _NBUF, s).wait()

    return emb_kernel


def kernel(x, pos_num, table):
    B, L = x.shape
    V, D = table.shape
    t = table.at[0].set(0.0)
    pe = jnp.asarray(_PE[:L])
    iota = jnp.arange(L, dtype=jnp.int32)
    nseq = B // _NW
    c1 = min(L, 128)
    x3 = x.reshape(_NW, nseq, L).astype(jnp.int32)
    xa = x3[:, :, :c1]
    xb = x3[:, :, c1:]
    out = _build(B, L, V, D)(xa, xb, pe, t, iota)
    return out.reshape(B, L, D)
